# baseline (speedup 1.0000x reference)
import functools

import jax
import jax.numpy as jnp
from jax import lax
from jax.experimental import pallas as pl
from jax.experimental.pallas import tpu as pltpu
from jax.experimental.pallas import tpu_sc as plsc

H = 450
HP = 512
HW = HP // 2
LAT = 28
LATP = 128
B = 256
NA = 10000
NAP = 10240
NB = 160000
MAXNB = 16
DA = 128
DB = 144

NC = 2
NS = 16
NW = NC * NS
BN = 8
NR = 3


def _pack_halves(r, relu=True):
    if relu:
        r = jnp.maximum(r, 0.0)
    rb = lax.bitcast_convert_type(r.astype(jnp.bfloat16), jnp.int16)
    lo = rb[:, :HW].astype(jnp.int32) & 0xFFFF
    hi = rb[:, HW:].astype(jnp.int32) & 0xFFFF
    return (hi << 16) | lo


def _unpack_halves(w):
    lo = lax.bitcast_convert_type(w << 16, jnp.float32)
    hi = lax.bitcast_convert_type(w & jnp.int32(-65536), jnp.float32)
    return jnp.concatenate([lo, hi], axis=1)


def _sc_gather_relu_sum(table, idx2d, row0, n_out, ch):
    per_w = n_out // NW
    nch = per_w // ch
    nbatch = ch // BN
    mesh = plsc.VectorSubcoreMesh(core_axis_name="c", subcore_axis_name="s")

    @functools.partial(
        pl.kernel,
        out_type=jax.ShapeDtypeStruct((n_out, HW), jnp.int32),
        mesh=mesh,
        compiler_params=pltpu.CompilerParams(needs_layout_passes=False),
        scratch_types=(
            [pltpu.VMEM((ch * MAXNB,), jnp.int32)]
            + [pltpu.VMEM((BN * MAXNB, HW), jnp.int32)] * NR
            + [pltpu.VMEM((BN, HW), jnp.int32)] * NR
            + [pltpu.SemaphoreType.DMA] * (2 * NR)
        ),
    )
    def k(tbl, idxf, out, idx_v, *bufs):
        rows = bufs[0:NR]
        obs = bufs[NR:2 * NR]
        sgs = bufs[2 * NR:3 * NR]
        sos = bufs[3 * NR:4 * NR]
        wid = lax.axis_index("s") * NC + lax.axis_index("c")
        base = wid * per_w

        def start_gather(bi, buf, sem):
            pltpu.async_copy(
                tbl.at[idx_v.at[pl.ds(bi * BN * MAXNB, BN * MAXNB)]], buf, sem)

        def do_batch(bi, kk, cbase, wait_prev_out):
            buf, sem, ob, so = rows[kk], sgs[kk], obs[kk], sos[kk]
            pltpu.make_async_copy(
                tbl.at[pl.ds(0, BN * MAXNB)], buf, sem).wait()

            def wait_out():
                pltpu.make_async_copy(
                    ob, out.at[pl.ds(cbase, BN)], so).wait()

            if isinstance(wait_prev_out, bool):
                if wait_prev_out:
                    wait_out()
            else:
                pl.when(wait_prev_out)(wait_out)

            def hloop(h, c3):
                for b in range(BN):
                    acc_lo = None
                    acc_hi = None
                    for nn in range(MAXNB):
                        w = buf[b * MAXNB + nn, pl.ds(h * 16, 16)]
                        lo = plsc.bitcast(w << 16, jnp.float32)
                        hi = plsc.bitcast(w & jnp.int32(-65536), jnp.float32)
                        acc_lo = lo if acc_lo is None else acc_lo + lo
                        acc_hi = hi if acc_hi is None else acc_hi + hi
                    u_lo = plsc.bitcast(acc_lo, jnp.int32)
                    u_hi = plsc.bitcast(acc_hi, jnp.int32)
                    r_lo = u_lo + 0x7FFF + ((u_lo >> 16) & 1)
                    r_hi = u_hi + 0x7FFF + ((u_hi >> 16) & 1)
                    ob[b, pl.ds(h * 16, 16)] = (
                        (r_hi & jnp.int32(-65536)) | ((r_lo >> 16) & 0xFFFF))
                return c3

            lax.fori_loop(0, HW // 16, hloop, 0)

            nxt = bi + NR < nbatch
            if isinstance(nxt, bool):
                if nxt:
                    start_gather(bi + NR, buf, sem)
            else:
                pl.when(nxt)(lambda: start_gather(bi + NR, buf, sem))

            pltpu.async_copy(ob, out.at[pl.ds(cbase + bi * BN, BN)], so)

        ngrp = nbatch // NR
        tail = nbatch % NR

        def chunk(ci, carry):
            cbase = base + ci * ch
            pltpu.sync_copy(
                idxf.at[pl.ds((row0 + cbase) * MAXNB, ch * MAXNB)], idx_v)
            for kk in range(NR):
                start_gather(kk, rows[kk], sgs[kk])

            def grp(g, c2):
                for kk in range(NR):
                    do_batch(g * NR + kk, kk, cbase, g > 0)
                return c2

            lax.fori_loop(0, ngrp, grp, 0)
            for t in range(tail):
                bi = ngrp * NR + t
                do_batch(bi, bi % NR, cbase, nbatch > NR)
            for kk in range(NR):
                pltpu.make_async_copy(
                    obs[kk], out.at[pl.ds(cbase, BN)], sos[kk]).wait()
            return carry

        lax.fori_loop(0, nch, chunk, 0)

    return k(table, idx2d)


def _tc_binput(fbonds, W_i_b):
    blk = 640

    def body(fb_ref, w_ref, o_ref, t_ref):
        r = jnp.dot(fb_ref[...].astype(jnp.bfloat16), w_ref[...],
                    preferred_element_type=jnp.float32)
        o_ref[...] = _pack_halves(r, relu=False)
        t_ref[...] = _pack_halves(r)

    return pl.pallas_call(
        body,
        grid=(NB // blk,),
        in_specs=[pl.BlockSpec((blk, DB), lambda i: (i, 0)),
                  pl.BlockSpec((DB, HP), lambda i: (0, 0))],
        out_specs=[pl.BlockSpec((blk, HW), lambda i: (i, 0)),
                   pl.BlockSpec((blk, HW), lambda i: (i, 0))],
        out_shape=[jax.ShapeDtypeStruct((NB, HW), jnp.int32),
                   jax.ShapeDtypeStruct((NB, HW), jnp.int32)],
    )(fbonds, W_i_b)


def _tc_update(binput, nei_c, W_h_p, row0, nrows, acc):
    blk = 640
    off = row0 // blk

    def body(b_ref, n_ref, w_ref, *rest):
        t_ref = rest[-1]
        nei_b = _unpack_halves(n_ref[...]).astype(jnp.bfloat16)
        r = _unpack_halves(b_ref[...]) + jnp.dot(
            nei_b, w_ref[...], preferred_element_type=jnp.float32)
        t_ref[...] = _pack_halves(r)

    in_specs = [pl.BlockSpec((blk, HW), lambda i: (i + off, 0)),
                pl.BlockSpec((blk, HW), lambda i: (i, 0)),
                pl.BlockSpec((HP, HP), lambda i: (0, 0))]
    args = [binput, nei_c, W_h_p]
    aliases = {}
    if acc is not None:
        in_specs.append(pl.BlockSpec(memory_space=pl.ANY))
        args.append(acc)
        aliases = {3: 0}
    return pl.pallas_call(
        body,
        grid=(nrows // blk,),
        in_specs=in_specs,
        out_specs=pl.BlockSpec((blk, HW), lambda i: (i + off, 0)),
        out_shape=jax.ShapeDtypeStruct((NB, HW), jnp.int32),
        input_output_aliases=aliases,
    )(*args)


def _tc_readout(fatoms, nei_a, mol3d, Wo_a, Wo_n):
    blk = 1000

    def body(fat_ref, nei_ref, mol_ref, wa_ref, wn_ref, sum_ref, cnt_ref):
        i = pl.program_id(0)
        nei_b = _unpack_halves(nei_ref[...]).astype(jnp.bfloat16)
        ah = jnp.maximum(
            jnp.dot(fat_ref[...], wa_ref[...],
                    preferred_element_type=jnp.float32)
            + jnp.dot(nei_b, wn_ref[...],
                      preferred_element_type=jnp.float32), 0.0)
        mids = mol_ref[0, 0, :]
        mask = (mids[:, None]
                == lax.broadcasted_iota(jnp.int32, (blk, B), 1)
                ).astype(jnp.float32)
        psum = lax.dot_general(mask, ah, (((0,), (0,)), ((), ())),
                               preferred_element_type=jnp.float32)
        pcnt = jnp.sum(mask, axis=0)[:, None]

        @pl.when(i == 0)
        def _():
            sum_ref[...] = jnp.zeros_like(sum_ref)
            cnt_ref[...] = jnp.zeros_like(cnt_ref)

        sum_ref[...] += psum
        cnt_ref[...] += pcnt

    return pl.pallas_call(
        body,
        grid=(NA // blk,),
        in_specs=[pl.BlockSpec((blk, DA), lambda i: (i, 0)),
                  pl.BlockSpec((blk, HW), lambda i: (i, 0)),
                  pl.BlockSpec((1, 1, blk), lambda i: (i, 0, 0)),
                  pl.BlockSpec((DA, HP), lambda i: (0, 0)),
                  pl.BlockSpec((HP, HP), lambda i: (0, 0))],
        out_specs=[pl.BlockSpec((B, HP), lambda i: (0, 0)),
                   pl.BlockSpec((B, 1), lambda i: (0, 0))],
        out_shape=[jax.ShapeDtypeStruct((B, HP), jnp.float32),
                   jax.ShapeDtypeStruct((B, 1), jnp.float32)],
    )(fatoms, nei_a, mol3d, Wo_a, Wo_n)


def _tc_tail(sums, cnts, Gm_w_p, Gm_b_p, Gv_w_p, Gv_b_p,
             F1_w_p, F1_b_p, F2_w_p, F2_b_p, features, beta_arr):
    def body(s_ref, c_ref, gmw, gmb, gvw, gvb, f1w, f1b, f2w, f2b,
             feat, beta_ref, o_ref):
        cnt = jnp.maximum(c_ref[...], 1.0)
        mv = s_ref[...] / cnt
        zm = jnp.dot(mv, gmw[...], preferred_element_type=jnp.float32) + gmb[...]
        zlv = -jnp.abs(
            jnp.dot(mv, gvw[...], preferred_element_type=jnp.float32) + gvb[...])
        kl = -0.5 * jnp.sum(1.0 + zlv - zm * zm - jnp.exp(zlv)) / B
        h1 = jnp.maximum(
            jnp.dot(mv, f1w[...], preferred_element_type=jnp.float32)
            + f1b[...], 0.0)
        pred = jnp.dot(h1, f2w[...], preferred_element_type=jnp.float32) + f2b[...]
        dp = pred[:, 0:1] - feat[...]
        prop = jnp.mean(dp * dp)
        o_ref[...] = prop + beta_ref[...] * kl

    return pl.pallas_call(
        body,
        out_shape=jax.ShapeDtypeStruct((1, 1), jnp.float32),
    )(sums, cnts, Gm_w_p, Gm_b_p, Gv_w_p, Gv_b_p,
      F1_w_p, F1_b_p, F2_w_p, F2_b_p, features, beta_arr)


def kernel(fatoms, fbonds, agraph, bgraph, mol_ids, features, epsilon, beta,
           W_i, W_h, W_o, Gm_w, Gm_b, Gv_w, Gv_b, F1_w, F1_b, F2_w, F2_b):
    f32 = jnp.float32
    bf16 = jnp.bfloat16
    W_i_b = jnp.pad(W_i, ((0, 0), (0, HP - H))).astype(bf16)
    W_h_b = jnp.pad(W_h, ((0, HP - H), (0, HP - H))).astype(bf16)
    Wo_a = jnp.pad(W_o[:DA], ((0, 0), (0, HP - H))).astype(bf16)
    Wo_n = jnp.pad(W_o[DA:], ((0, HP - H), (0, HP - H))).astype(bf16)
    Gm_w_p = jnp.pad(Gm_w, ((0, HP - H), (0, LATP - LAT)))
    Gm_b_p = jnp.pad(Gm_b, ((0, LATP - LAT)))[None, :]
    Gv_w_p = jnp.pad(Gv_w, ((0, HP - H), (0, LATP - LAT)))
    Gv_b_p = jnp.pad(Gv_b, ((0, LATP - LAT)))[None, :]
    F1_w_p = jnp.pad(F1_w, ((0, HP - H), (0, HP - H)))
    F1_b_p = jnp.pad(F1_b, ((0, HP - H)))[None, :]
    F2_w_p = jnp.pad(F2_w, ((0, HP - H), (0, LATP - 1)))
    F2_b_p = jnp.pad(F2_b, ((0, LATP - 1)))[None, :]
    beta_arr = jnp.asarray(beta, f32).reshape(1, 1)

    bgraph_flat = bgraph.astype(jnp.int32).reshape(-1)
    agraph_flat = jnp.pad(agraph.astype(jnp.int32),
                          ((0, NAP - NA), (0, 0))).reshape(-1)
    mol3d = mol_ids.astype(jnp.int32).reshape(NA // 1000, 1, 1000)

    binput, btable = _tc_binput(fbonds, W_i_b)
    sizes = (51200, 46080, 35840, 20480, 6400)
    offs = [sum(sizes[:c]) for c in range(len(sizes))]

    def mp_round(table):
        neis = [
            _sc_gather_relu_sum(table, bgraph_flat, offs[c], sizes[c],
                                sizes[c] // NW)
            for c in range(len(sizes))]
        acc = None
        for c in range(len(sizes)):
            acc = _tc_update(binput, neis[c], W_h_b, offs[c], sizes[c], acc)
        return acc

    m1 = mp_round(btable)
    m2 = mp_round(m1)
    nei_a = _sc_gather_relu_sum(m2, agraph_flat, 0, NAP, NAP // NW)
    sums, cnts = _tc_readout(fatoms.astype(bf16), nei_a, mol3d, Wo_a, Wo_n)
    out = _tc_tail(sums, cnts, Gm_w_p, Gm_b_p, Gv_w_p, Gv_b_p,
                   F1_w_p, F1_b_p, F2_w_p, F2_b_p, features, beta_arr)
    return out.reshape(())


# speedup vs baseline: 1.0270x; 1.0270x over previous
import functools

import jax
import jax.numpy as jnp
from jax import lax
from jax.experimental import pallas as pl
from jax.experimental.pallas import tpu as pltpu
from jax.experimental.pallas import tpu_sc as plsc

H = 450
HP = 512
HW = HP // 2
LAT = 28
LATP = 128
B = 256
NA = 10000
NAP = 10240
NB = 160000
MAXNB = 16
DA = 128
DB = 144

NC = 2
NS = 16
NW = NC * NS
BN = 8
NR = 3


def _pack_halves(r, relu=True):
    if relu:
        r = jnp.maximum(r, 0.0)
    rb = lax.bitcast_convert_type(r.astype(jnp.bfloat16), jnp.int16)
    lo = rb[:, :HW].astype(jnp.int32) & 0xFFFF
    hi = rb[:, HW:].astype(jnp.int32) & 0xFFFF
    return (hi << 16) | lo


def _unpack_halves(w):
    lo = lax.bitcast_convert_type(w << 16, jnp.float32)
    hi = lax.bitcast_convert_type(w & jnp.int32(-65536), jnp.float32)
    return jnp.concatenate([lo, hi], axis=1)


def _sc_gather_relu_sum(table, idx2d, row0, n_out, ch):
    per_w = n_out // NW
    nch = per_w // ch
    nbatch = ch // BN
    mesh = plsc.VectorSubcoreMesh(core_axis_name="c", subcore_axis_name="s")

    @functools.partial(
        pl.kernel,
        out_type=jax.ShapeDtypeStruct((n_out, HW), jnp.int32),
        mesh=mesh,
        compiler_params=pltpu.CompilerParams(needs_layout_passes=False),
        scratch_types=(
            [pltpu.VMEM((ch * MAXNB,), jnp.int32)]
            + [pltpu.VMEM((BN * MAXNB, HW), jnp.int32)] * NR
            + [pltpu.VMEM((BN, HW), jnp.int32)] * NR
            + [pltpu.SemaphoreType.DMA] * (2 * NR)
        ),
    )
    def k(tbl, idxf, out, idx_v, *bufs):
        rows = bufs[0:NR]
        obs = bufs[NR:2 * NR]
        sgs = bufs[2 * NR:3 * NR]
        sos = bufs[3 * NR:4 * NR]
        wid = lax.axis_index("s") * NC + lax.axis_index("c")
        base = wid * per_w

        def start_gather(bi, buf, sem):
            pltpu.async_copy(
                tbl.at[idx_v.at[pl.ds(bi * BN * MAXNB, BN * MAXNB)]], buf, sem)

        def do_batch(bi, kk, cbase, wait_prev_out):
            buf, sem, ob, so = rows[kk], sgs[kk], obs[kk], sos[kk]
            pltpu.make_async_copy(
                tbl.at[pl.ds(0, BN * MAXNB)], buf, sem).wait()

            def wait_out():
                pltpu.make_async_copy(
                    ob, out.at[pl.ds(cbase, BN)], so).wait()

            if isinstance(wait_prev_out, bool):
                if wait_prev_out:
                    wait_out()
            else:
                pl.when(wait_prev_out)(wait_out)

            def hloop(h, c3):
                for b in range(BN):
                    acc_lo = None
                    acc_hi = None
                    for nn in range(MAXNB):
                        w = buf[b * MAXNB + nn, pl.ds(h * 16, 16)]
                        lo = plsc.bitcast(w << 16, jnp.float32)
                        hi = plsc.bitcast(w & jnp.int32(-65536), jnp.float32)
                        acc_lo = lo if acc_lo is None else acc_lo + lo
                        acc_hi = hi if acc_hi is None else acc_hi + hi
                    u_lo = plsc.bitcast(acc_lo, jnp.int32)
                    u_hi = plsc.bitcast(acc_hi, jnp.int32)
                    r_lo = u_lo + 0x7FFF + ((u_lo >> 16) & 1)
                    r_hi = u_hi + 0x7FFF + ((u_hi >> 16) & 1)
                    ob[b, pl.ds(h * 16, 16)] = (
                        (r_hi & jnp.int32(-65536)) | ((r_lo >> 16) & 0xFFFF))
                return c3

            lax.fori_loop(0, HW // 16, hloop, 0)

            nxt = bi + NR < nbatch
            if isinstance(nxt, bool):
                if nxt:
                    start_gather(bi + NR, buf, sem)
            else:
                pl.when(nxt)(lambda: start_gather(bi + NR, buf, sem))

            pltpu.async_copy(ob, out.at[pl.ds(cbase + bi * BN, BN)], so)

        ngrp = nbatch // NR
        tail = nbatch % NR

        def chunk(ci, carry):
            cbase = base + ci * ch
            pltpu.sync_copy(
                idxf.at[pl.ds((row0 + cbase) * MAXNB, ch * MAXNB)], idx_v)
            for kk in range(NR):
                start_gather(kk, rows[kk], sgs[kk])

            def grp(g, c2):
                for kk in range(NR):
                    do_batch(g * NR + kk, kk, cbase, g > 0)
                return c2

            lax.fori_loop(0, ngrp, grp, 0)
            for t in range(tail):
                bi = ngrp * NR + t
                do_batch(bi, bi % NR, cbase, nbatch > NR)
            for kk in range(NR):
                pltpu.make_async_copy(
                    obs[kk], out.at[pl.ds(cbase, BN)], sos[kk]).wait()
            return carry

        lax.fori_loop(0, nch, chunk, 0)

    return k(table, idx2d)


def _tc_binput(fbonds_b, W_i_b):
    blk = 640

    def body(fb_ref, w_ref, t_ref):
        r = jnp.dot(fb_ref[...], w_ref[...], preferred_element_type=jnp.float32)
        t_ref[...] = _pack_halves(r)

    return pl.pallas_call(
        body,
        grid=(NB // blk,),
        in_specs=[pl.BlockSpec((blk, DB), lambda i: (i, 0)),
                  pl.BlockSpec((DB, HP), lambda i: (0, 0))],
        out_specs=pl.BlockSpec((blk, HW), lambda i: (i, 0)),
        out_shape=jax.ShapeDtypeStruct((NB, HW), jnp.int32),
    )(fbonds_b, W_i_b)


def _tc_update(fbonds_b, W_i_b, nei_c, W_h_p, row0, nrows, acc):
    blk = 640
    off = row0 // blk

    def body(fb_ref, wi_ref, n_ref, w_ref, *rest):
        t_ref = rest[-1]
        nei_b = _unpack_halves(n_ref[...]).astype(jnp.bfloat16)
        r = (jnp.dot(fb_ref[...], wi_ref[...],
                     preferred_element_type=jnp.float32)
             + jnp.dot(nei_b, w_ref[...],
                       preferred_element_type=jnp.float32))
        t_ref[...] = _pack_halves(r)

    in_specs = [pl.BlockSpec((blk, DB), lambda i: (i + off, 0)),
                pl.BlockSpec((DB, HP), lambda i: (0, 0)),
                pl.BlockSpec((blk, HW), lambda i: (i, 0)),
                pl.BlockSpec((HP, HP), lambda i: (0, 0))]
    args = [fbonds_b, W_i_b, nei_c, W_h_p]
    aliases = {}
    if acc is not None:
        in_specs.append(pl.BlockSpec(memory_space=pl.ANY))
        args.append(acc)
        aliases = {4: 0}
    return pl.pallas_call(
        body,
        grid=(nrows // blk,),
        in_specs=in_specs,
        out_specs=pl.BlockSpec((blk, HW), lambda i: (i + off, 0)),
        out_shape=jax.ShapeDtypeStruct((NB, HW), jnp.int32),
        input_output_aliases=aliases,
    )(*args)


def _tc_readout(fatoms, nei_a, mol3d, Wo_a, Wo_n):
    blk = 1000

    def body(fat_ref, nei_ref, mol_ref, wa_ref, wn_ref, sum_ref, cnt_ref):
        i = pl.program_id(0)
        nei_b = _unpack_halves(nei_ref[...]).astype(jnp.bfloat16)
        ah = jnp.maximum(
            jnp.dot(fat_ref[...], wa_ref[...],
                    preferred_element_type=jnp.float32)
            + jnp.dot(nei_b, wn_ref[...],
                      preferred_element_type=jnp.float32), 0.0)
        mids = mol_ref[0, 0, :]
        mask = (mids[:, None]
                == lax.broadcasted_iota(jnp.int32, (blk, B), 1)
                ).astype(jnp.float32)
        psum = lax.dot_general(mask, ah, (((0,), (0,)), ((), ())),
                               preferred_element_type=jnp.float32)
        pcnt = jnp.sum(mask, axis=0)[:, None]

        @pl.when(i == 0)
        def _():
            sum_ref[...] = jnp.zeros_like(sum_ref)
            cnt_ref[...] = jnp.zeros_like(cnt_ref)

        sum_ref[...] += psum
        cnt_ref[...] += pcnt

    return pl.pallas_call(
        body,
        grid=(NA // blk,),
        in_specs=[pl.BlockSpec((blk, DA), lambda i: (i, 0)),
                  pl.BlockSpec((blk, HW), lambda i: (i, 0)),
                  pl.BlockSpec((1, 1, blk), lambda i: (i, 0, 0)),
                  pl.BlockSpec((DA, HP), lambda i: (0, 0)),
                  pl.BlockSpec((HP, HP), lambda i: (0, 0))],
        out_specs=[pl.BlockSpec((B, HP), lambda i: (0, 0)),
                   pl.BlockSpec((B, 1), lambda i: (0, 0))],
        out_shape=[jax.ShapeDtypeStruct((B, HP), jnp.float32),
                   jax.ShapeDtypeStruct((B, 1), jnp.float32)],
    )(fatoms, nei_a, mol3d, Wo_a, Wo_n)


def _tc_tail(sums, cnts, Gm_w_p, Gm_b_p, Gv_w_p, Gv_b_p,
             F1_w_p, F1_b_p, F2_w_p, F2_b_p, features, beta_arr):
    def body(s_ref, c_ref, gmw, gmb, gvw, gvb, f1w, f1b, f2w, f2b,
             feat, beta_ref, o_ref):
        cnt = jnp.maximum(c_ref[...], 1.0)
        mv = s_ref[...] / cnt
        zm = jnp.dot(mv, gmw[...], preferred_element_type=jnp.float32) + gmb[...]
        zlv = -jnp.abs(
            jnp.dot(mv, gvw[...], preferred_element_type=jnp.float32) + gvb[...])
        kl = -0.5 * jnp.sum(1.0 + zlv - zm * zm - jnp.exp(zlv)) / B
        h1 = jnp.maximum(
            jnp.dot(mv, f1w[...], preferred_element_type=jnp.float32)
            + f1b[...], 0.0)
        pred = jnp.dot(h1, f2w[...], preferred_element_type=jnp.float32) + f2b[...]
        dp = pred[:, 0:1] - feat[...]
        prop = jnp.mean(dp * dp)
        o_ref[...] = prop + beta_ref[...] * kl

    return pl.pallas_call(
        body,
        out_shape=jax.ShapeDtypeStruct((1, 1), jnp.float32),
    )(sums, cnts, Gm_w_p, Gm_b_p, Gv_w_p, Gv_b_p,
      F1_w_p, F1_b_p, F2_w_p, F2_b_p, features, beta_arr)


def kernel(fatoms, fbonds, agraph, bgraph, mol_ids, features, epsilon, beta,
           W_i, W_h, W_o, Gm_w, Gm_b, Gv_w, Gv_b, F1_w, F1_b, F2_w, F2_b):
    f32 = jnp.float32
    bf16 = jnp.bfloat16
    W_i_b = jnp.pad(W_i, ((0, 0), (0, HP - H))).astype(bf16)
    W_h_b = jnp.pad(W_h, ((0, HP - H), (0, HP - H))).astype(bf16)
    Wo_a = jnp.pad(W_o[:DA], ((0, 0), (0, HP - H))).astype(bf16)
    Wo_n = jnp.pad(W_o[DA:], ((0, HP - H), (0, HP - H))).astype(bf16)
    Gm_w_p = jnp.pad(Gm_w, ((0, HP - H), (0, LATP - LAT)))
    Gm_b_p = jnp.pad(Gm_b, ((0, LATP - LAT)))[None, :]
    Gv_w_p = jnp.pad(Gv_w, ((0, HP - H), (0, LATP - LAT)))
    Gv_b_p = jnp.pad(Gv_b, ((0, LATP - LAT)))[None, :]
    F1_w_p = jnp.pad(F1_w, ((0, HP - H), (0, HP - H)))
    F1_b_p = jnp.pad(F1_b, ((0, HP - H)))[None, :]
    F2_w_p = jnp.pad(F2_w, ((0, HP - H), (0, LATP - 1)))
    F2_b_p = jnp.pad(F2_b, ((0, LATP - 1)))[None, :]
    beta_arr = jnp.asarray(beta, f32).reshape(1, 1)

    bgraph_flat = bgraph.astype(jnp.int32).reshape(-1)
    agraph_flat = jnp.pad(agraph.astype(jnp.int32),
                          ((0, NAP - NA), (0, 0))).reshape(-1)
    mol3d = mol_ids.astype(jnp.int32).reshape(NA // 1000, 1, 1000)

    fbonds_b = fbonds.astype(bf16)
    btable = _tc_binput(fbonds_b, W_i_b)
    sizes = (51200, 46080, 35840, 20480, 6400)
    offs = [sum(sizes[:c]) for c in range(len(sizes))]

    def mp_round(table):
        neis = [
            _sc_gather_relu_sum(table, bgraph_flat, offs[c], sizes[c],
                                sizes[c] // NW)
            for c in range(len(sizes))]
        acc = None
        for c in range(len(sizes)):
            acc = _tc_update(fbonds_b, W_i_b, neis[c], W_h_b,
                             offs[c], sizes[c], acc)
        return acc

    m1 = mp_round(btable)
    m2 = mp_round(m1)
    nei_a = _sc_gather_relu_sum(m2, agraph_flat, 0, NAP, NAP // NW)
    sums, cnts = _tc_readout(fatoms.astype(bf16), nei_a, mol3d, Wo_a, Wo_n)
    out = _tc_tail(sums, cnts, Gm_w_p, Gm_b_p, Gv_w_p, Gv_b_p,
                   F1_w_p, F1_b_p, F2_w_p, F2_b_p, features, beta_arr)
    return out.reshape(())


# speedup vs baseline: 1.0881x; 1.0595x over previous
import functools

import jax
import jax.numpy as jnp
from jax import lax
from jax.experimental import pallas as pl
from jax.experimental.pallas import tpu as pltpu
from jax.experimental.pallas import tpu_sc as plsc

H = 450
HP = 512
HW = HP // 2
LAT = 28
LATP = 128
B = 256
NA = 10000
NAP = 10240
NB = 160000
MAXNB = 16
DA = 128
DB = 144

NC = 2
NS = 16
NW = NC * NS
BN = 8
NR = 3


def _pack_halves(r, relu=True):
    if relu:
        r = jnp.maximum(r, 0.0)
    rb = lax.bitcast_convert_type(r.astype(jnp.bfloat16), jnp.int16)
    lo = rb[:, :HW].astype(jnp.int32) & 0xFFFF
    hi = rb[:, HW:].astype(jnp.int32) & 0xFFFF
    return (hi << 16) | lo


def _unpack_halves(w):
    lo = lax.bitcast_convert_type(w << 16, jnp.float32)
    hi = lax.bitcast_convert_type(w & jnp.int32(-65536), jnp.float32)
    return jnp.concatenate([lo, hi], axis=1)


def _sc_gather_relu_sum(table, idx2d, row0, n_out, ch):
    per_w = n_out // NW
    nch = per_w // ch
    nbatch = ch // BN
    mesh = plsc.VectorSubcoreMesh(core_axis_name="c", subcore_axis_name="s")

    @functools.partial(
        pl.kernel,
        out_type=jax.ShapeDtypeStruct((n_out, HW), jnp.int32),
        mesh=mesh,
        compiler_params=pltpu.CompilerParams(needs_layout_passes=False),
        scratch_types=(
            [pltpu.VMEM((ch * MAXNB,), jnp.int32)]
            + [pltpu.VMEM((BN * MAXNB, HW), jnp.int32)] * NR
            + [pltpu.VMEM((BN, HW), jnp.int32)] * NR
            + [pltpu.SemaphoreType.DMA] * (2 * NR)
        ),
    )
    def k(tbl, idxf, out, idx_v, *bufs):
        rows = bufs[0:NR]
        obs = bufs[NR:2 * NR]
        sgs = bufs[2 * NR:3 * NR]
        sos = bufs[3 * NR:4 * NR]
        wid = lax.axis_index("s") * NC + lax.axis_index("c")
        base = wid * per_w

        def start_gather(bi, buf, sem):
            pltpu.async_copy(
                tbl.at[idx_v.at[pl.ds(bi * BN * MAXNB, BN * MAXNB)]], buf, sem)

        def do_batch(bi, kk, cbase, wait_prev_out):
            buf, sem, ob, so = rows[kk], sgs[kk], obs[kk], sos[kk]
            pltpu.make_async_copy(
                tbl.at[pl.ds(0, BN * MAXNB)], buf, sem).wait()

            def wait_out():
                pltpu.make_async_copy(
                    ob, out.at[pl.ds(cbase, BN)], so).wait()

            if isinstance(wait_prev_out, bool):
                if wait_prev_out:
                    wait_out()
            else:
                pl.when(wait_prev_out)(wait_out)

            def hloop(h, c3):
                for b in range(BN):
                    acc_lo = None
                    acc_hi = None
                    for nn in range(MAXNB):
                        w = buf[b * MAXNB + nn, pl.ds(h * 16, 16)]
                        lo = plsc.bitcast(w << 16, jnp.float32)
                        hi = plsc.bitcast(w & jnp.int32(-65536), jnp.float32)
                        acc_lo = lo if acc_lo is None else acc_lo + lo
                        acc_hi = hi if acc_hi is None else acc_hi + hi
                    u_lo = plsc.bitcast(acc_lo, jnp.int32)
                    u_hi = plsc.bitcast(acc_hi, jnp.int32)
                    r_lo = u_lo + 0x7FFF + ((u_lo >> 16) & 1)
                    r_hi = u_hi + 0x7FFF + ((u_hi >> 16) & 1)
                    ob[b, pl.ds(h * 16, 16)] = (
                        (r_hi & jnp.int32(-65536)) | ((r_lo >> 16) & 0xFFFF))
                return c3

            lax.fori_loop(0, HW // 16, hloop, 0)

            nxt = bi + NR < nbatch
            if isinstance(nxt, bool):
                if nxt:
                    start_gather(bi + NR, buf, sem)
            else:
                pl.when(nxt)(lambda: start_gather(bi + NR, buf, sem))

            pltpu.async_copy(ob, out.at[pl.ds(cbase + bi * BN, BN)], so)

        ngrp = nbatch // NR
        tail = nbatch % NR

        def chunk(ci, carry):
            cbase = base + ci * ch
            pltpu.sync_copy(
                idxf.at[pl.ds((row0 + cbase) * MAXNB, ch * MAXNB)], idx_v)
            for kk in range(NR):
                start_gather(kk, rows[kk], sgs[kk])

            def grp(g, c2):
                for kk in range(NR):
                    do_batch(g * NR + kk, kk, cbase, g > 0)
                return c2

            lax.fori_loop(0, ngrp, grp, 0)
            for t in range(tail):
                bi = ngrp * NR + t
                do_batch(bi, bi % NR, cbase, nbatch > NR)
            for kk in range(NR):
                pltpu.make_async_copy(
                    obs[kk], out.at[pl.ds(cbase, BN)], sos[kk]).wait()
            return carry

        lax.fori_loop(0, nch, chunk, 0)

    return k(table, idx2d)


def _tc_binput(fbonds_b, W_i_b):
    blk = 640

    def body(fb_ref, w_ref, t_ref):
        r = jnp.dot(fb_ref[...], w_ref[...], preferred_element_type=jnp.float32)
        t_ref[...] = _pack_halves(r)

    return pl.pallas_call(
        body,
        grid=(NB // blk,),
        in_specs=[pl.BlockSpec((blk, DB), lambda i: (i, 0)),
                  pl.BlockSpec((DB, HP), lambda i: (0, 0))],
        out_specs=pl.BlockSpec((blk, HW), lambda i: (i, 0)),
        out_shape=jax.ShapeDtypeStruct((NB, HW), jnp.int32),
    )(fbonds_b, W_i_b)


def _tc_update(fbonds_b, W_i_b, nei_c, W_h_p, row0, nrows, acc):
    blk = 640
    off = row0 // blk

    def body(fb_ref, wi_ref, n_ref, w_ref, *rest):
        t_ref = rest[-1]
        nei_b = _unpack_halves(n_ref[...]).astype(jnp.bfloat16)
        r = (jnp.dot(fb_ref[...], wi_ref[...],
                     preferred_element_type=jnp.float32)
             + jnp.dot(nei_b, w_ref[...],
                       preferred_element_type=jnp.float32))
        t_ref[...] = _pack_halves(r)

    in_specs = [pl.BlockSpec((blk, DB), lambda i: (i + off, 0)),
                pl.BlockSpec((DB, HP), lambda i: (0, 0)),
                pl.BlockSpec((blk, HW), lambda i: (i, 0)),
                pl.BlockSpec((HP, HP), lambda i: (0, 0))]
    args = [fbonds_b, W_i_b, nei_c, W_h_p]
    aliases = {}
    if acc is not None:
        in_specs.append(pl.BlockSpec(memory_space=pl.ANY))
        args.append(acc)
        aliases = {4: 0}
    return pl.pallas_call(
        body,
        grid=(nrows // blk,),
        in_specs=in_specs,
        out_specs=pl.BlockSpec((blk, HW), lambda i: (i + off, 0)),
        out_shape=jax.ShapeDtypeStruct((NB, HW), jnp.int32),
        input_output_aliases=aliases,
    )(*args)


def _tc_readout(fatoms, nei_a, mol3d, Wo_a, Wo_n):
    blk = 1000

    def body(fat_ref, nei_ref, mol_ref, wa_ref, wn_ref, sum_ref, cnt_ref):
        i = pl.program_id(0)
        nei_b = _unpack_halves(nei_ref[...]).astype(jnp.bfloat16)
        ah = jnp.maximum(
            jnp.dot(fat_ref[...], wa_ref[...],
                    preferred_element_type=jnp.float32)
            + jnp.dot(nei_b, wn_ref[...],
                      preferred_element_type=jnp.float32), 0.0)
        mids = mol_ref[0, 0, :]
        mask = (mids[:, None]
                == lax.broadcasted_iota(jnp.int32, (blk, B), 1)
                ).astype(jnp.float32)
        psum = lax.dot_general(mask, ah, (((0,), (0,)), ((), ())),
                               preferred_element_type=jnp.float32)
        pcnt = jnp.sum(mask, axis=0)[:, None]

        @pl.when(i == 0)
        def _():
            sum_ref[...] = jnp.zeros_like(sum_ref)
            cnt_ref[...] = jnp.zeros_like(cnt_ref)

        sum_ref[...] += psum
        cnt_ref[...] += pcnt

    return pl.pallas_call(
        body,
        grid=(NA // blk,),
        in_specs=[pl.BlockSpec((blk, DA), lambda i: (i, 0)),
                  pl.BlockSpec((blk, HW), lambda i: (i, 0)),
                  pl.BlockSpec((1, 1, blk), lambda i: (i, 0, 0)),
                  pl.BlockSpec((DA, HP), lambda i: (0, 0)),
                  pl.BlockSpec((HP, HP), lambda i: (0, 0))],
        out_specs=[pl.BlockSpec((B, HP), lambda i: (0, 0)),
                   pl.BlockSpec((B, 1), lambda i: (0, 0))],
        out_shape=[jax.ShapeDtypeStruct((B, HP), jnp.float32),
                   jax.ShapeDtypeStruct((B, 1), jnp.float32)],
    )(fatoms, nei_a, mol3d, Wo_a, Wo_n)


def _tc_tail(sums, cnts, Gm_w_p, Gm_b_p, Gv_w_p, Gv_b_p,
             F1_w_p, F1_b_p, F2_w_p, F2_b_p, features, beta_arr):
    def body(s_ref, c_ref, gmw, gmb, gvw, gvb, f1w, f1b, f2w, f2b,
             feat, beta_ref, o_ref):
        cnt = jnp.maximum(c_ref[...], 1.0)
        mv = s_ref[...] / cnt
        zm = jnp.dot(mv, gmw[...], preferred_element_type=jnp.float32) + gmb[...]
        zlv = -jnp.abs(
            jnp.dot(mv, gvw[...], preferred_element_type=jnp.float32) + gvb[...])
        kl = -0.5 * jnp.sum(1.0 + zlv - zm * zm - jnp.exp(zlv)) / B
        h1 = jnp.maximum(
            jnp.dot(mv, f1w[...], preferred_element_type=jnp.float32)
            + f1b[...], 0.0)
        pred = jnp.dot(h1, f2w[...], preferred_element_type=jnp.float32) + f2b[...]
        dp = pred[:, 0:1] - feat[...]
        prop = jnp.mean(dp * dp)
        o_ref[...] = prop + beta_ref[...] * kl

    return pl.pallas_call(
        body,
        out_shape=jax.ShapeDtypeStruct((1, 1), jnp.float32),
    )(sums, cnts, Gm_w_p, Gm_b_p, Gv_w_p, Gv_b_p,
      F1_w_p, F1_b_p, F2_w_p, F2_b_p, features, beta_arr)


def kernel(fatoms, fbonds, agraph, bgraph, mol_ids, features, epsilon, beta,
           W_i, W_h, W_o, Gm_w, Gm_b, Gv_w, Gv_b, F1_w, F1_b, F2_w, F2_b):
    f32 = jnp.float32
    bf16 = jnp.bfloat16
    W_i_b = jnp.pad(W_i, ((0, 0), (0, HP - H))).astype(bf16)
    W_h_b = jnp.pad(W_h, ((0, HP - H), (0, HP - H))).astype(bf16)
    Wo_a = jnp.pad(W_o[:DA], ((0, 0), (0, HP - H))).astype(bf16)
    Wo_n = jnp.pad(W_o[DA:], ((0, HP - H), (0, HP - H))).astype(bf16)
    Gm_w_p = jnp.pad(Gm_w, ((0, HP - H), (0, LATP - LAT)))
    Gm_b_p = jnp.pad(Gm_b, ((0, LATP - LAT)))[None, :]
    Gv_w_p = jnp.pad(Gv_w, ((0, HP - H), (0, LATP - LAT)))
    Gv_b_p = jnp.pad(Gv_b, ((0, LATP - LAT)))[None, :]
    F1_w_p = jnp.pad(F1_w, ((0, HP - H), (0, HP - H)))
    F1_b_p = jnp.pad(F1_b, ((0, HP - H)))[None, :]
    F2_w_p = jnp.pad(F2_w, ((0, HP - H), (0, LATP - 1)))
    F2_b_p = jnp.pad(F2_b, ((0, LATP - 1)))[None, :]
    beta_arr = jnp.asarray(beta, f32).reshape(1, 1)

    bgraph_flat = bgraph.astype(jnp.int32).reshape(-1)
    pad_idx = (jnp.arange((NAP - NA) * MAXNB, dtype=jnp.int32)
               % NB).reshape(NAP - NA, MAXNB)
    agraph_flat = jnp.concatenate(
        [agraph.astype(jnp.int32), pad_idx], axis=0).reshape(-1)
    mol3d = mol_ids.astype(jnp.int32).reshape(NA // 1000, 1, 1000)

    fbonds_b = fbonds.astype(bf16)
    btable = _tc_binput(fbonds_b, W_i_b)
    sizes = (51200, 46080, 35840, 20480, 6400)
    offs = [sum(sizes[:c]) for c in range(len(sizes))]

    def mp_round(table):
        neis = [
            _sc_gather_relu_sum(table, bgraph_flat, offs[c], sizes[c],
                                sizes[c] // NW)
            for c in range(len(sizes))]
        acc = None
        for c in range(len(sizes)):
            acc = _tc_update(fbonds_b, W_i_b, neis[c], W_h_b,
                             offs[c], sizes[c], acc)
        return acc

    m1 = mp_round(btable)
    m2 = mp_round(m1)
    nei_a = _sc_gather_relu_sum(m2, agraph_flat, 0, NAP, NAP // NW)
    sums, cnts = _tc_readout(fatoms.astype(bf16), nei_a, mol3d, Wo_a, Wo_n)
    out = _tc_tail(sums, cnts, Gm_w_p, Gm_b_p, Gv_w_p, Gv_b_p,
                   F1_w_p, F1_b_p, F2_w_p, F2_b_p, features, beta_arr)
    return out.reshape(())


# speedup vs baseline: 1.0906x; 1.0023x over previous
import functools

import jax
import jax.numpy as jnp
from jax import lax
from jax.experimental import pallas as pl
from jax.experimental.pallas import tpu as pltpu
from jax.experimental.pallas import tpu_sc as plsc

H = 450
HP = 512
HW = HP // 2
LAT = 28
LATP = 128
B = 256
NA = 10000
NAP = 10240
NB = 160000
MAXNB = 16
DA = 128
DB = 144

NC = 2
NS = 16
NW = NC * NS
BN = 8
NR = 3


def _pack_halves(r, relu=True):
    if relu:
        r = jnp.maximum(r, 0.0)
    rb = lax.bitcast_convert_type(r.astype(jnp.bfloat16), jnp.int16)
    lo = rb[:, :HW].astype(jnp.int32) & 0xFFFF
    hi = rb[:, HW:].astype(jnp.int32) & 0xFFFF
    return (hi << 16) | lo


def _unpack_halves(w):
    lo = lax.bitcast_convert_type(w << 16, jnp.float32)
    hi = lax.bitcast_convert_type(w & jnp.int32(-65536), jnp.float32)
    return jnp.concatenate([lo, hi], axis=1)


def _sc_gather_relu_sum(table, idx2d, row0, n_out, ch):
    per_w = n_out // NW
    nch = per_w // ch
    nbatch = ch // BN
    mesh = plsc.VectorSubcoreMesh(core_axis_name="c", subcore_axis_name="s")

    @functools.partial(
        pl.kernel,
        out_type=jax.ShapeDtypeStruct((n_out, HW), jnp.int32),
        mesh=mesh,
        compiler_params=pltpu.CompilerParams(needs_layout_passes=False),
        scratch_types=(
            [pltpu.VMEM((ch * MAXNB,), jnp.int32)]
            + [pltpu.VMEM((BN * MAXNB, HW), jnp.int32)] * NR
            + [pltpu.VMEM((BN, HW), jnp.int32)] * NR
            + [pltpu.SemaphoreType.DMA] * (2 * NR)
        ),
    )
    def k(tbl, idxf, out, idx_v, *bufs):
        rows = bufs[0:NR]
        obs = bufs[NR:2 * NR]
        sgs = bufs[2 * NR:3 * NR]
        sos = bufs[3 * NR:4 * NR]
        wid = lax.axis_index("s") * NC + lax.axis_index("c")
        base = wid * per_w

        def start_gather(bi, buf, sem):
            pltpu.async_copy(
                tbl.at[idx_v.at[pl.ds(bi * BN * MAXNB, BN * MAXNB)]], buf, sem)

        def do_batch(bi, kk, cbase, wait_prev_out):
            buf, sem, ob, so = rows[kk], sgs[kk], obs[kk], sos[kk]
            pltpu.make_async_copy(
                tbl.at[pl.ds(0, BN * MAXNB)], buf, sem).wait()

            def wait_out():
                pltpu.make_async_copy(
                    ob, out.at[pl.ds(cbase, BN)], so).wait()

            if isinstance(wait_prev_out, bool):
                if wait_prev_out:
                    wait_out()
            else:
                pl.when(wait_prev_out)(wait_out)

            def hloop(h, c3):
                for b in range(BN):
                    acc_lo = None
                    acc_hi = None
                    for nn in range(MAXNB):
                        w = buf[b * MAXNB + nn, pl.ds(h * 16, 16)]
                        lo = plsc.bitcast(w << 16, jnp.float32)
                        hi = plsc.bitcast(w & jnp.int32(-65536), jnp.float32)
                        acc_lo = lo if acc_lo is None else acc_lo + lo
                        acc_hi = hi if acc_hi is None else acc_hi + hi
                    u_lo = plsc.bitcast(acc_lo, jnp.int32)
                    u_hi = plsc.bitcast(acc_hi, jnp.int32)
                    r_lo = u_lo + 0x7FFF + ((u_lo >> 16) & 1)
                    r_hi = u_hi + 0x7FFF + ((u_hi >> 16) & 1)
                    ob[b, pl.ds(h * 16, 16)] = (
                        (r_hi & jnp.int32(-65536)) | ((r_lo >> 16) & 0xFFFF))
                return c3

            lax.fori_loop(0, HW // 16, hloop, 0)

            nxt = bi + NR < nbatch
            if isinstance(nxt, bool):
                if nxt:
                    start_gather(bi + NR, buf, sem)
            else:
                pl.when(nxt)(lambda: start_gather(bi + NR, buf, sem))

            pltpu.async_copy(ob, out.at[pl.ds(cbase + bi * BN, BN)], so)

        ngrp = nbatch // NR
        tail = nbatch % NR

        def chunk(ci, carry):
            cbase = base + ci * ch
            pltpu.sync_copy(
                idxf.at[pl.ds((row0 + cbase) * MAXNB, ch * MAXNB)], idx_v)
            for kk in range(NR):
                start_gather(kk, rows[kk], sgs[kk])

            def grp(g, c2):
                for kk in range(NR):
                    do_batch(g * NR + kk, kk, cbase, g > 0)
                return c2

            lax.fori_loop(0, ngrp, grp, 0)
            for t in range(tail):
                bi = ngrp * NR + t
                do_batch(bi, bi % NR, cbase, nbatch > NR)
            for kk in range(NR):
                pltpu.make_async_copy(
                    obs[kk], out.at[pl.ds(cbase, BN)], sos[kk]).wait()
            return carry

        lax.fori_loop(0, nch, chunk, 0)

    return k(table, idx2d)


def _tc_binput(fbonds_b, W_i_b):
    blk = 640

    def body(fb_ref, w_ref, t_ref):
        r = jnp.dot(fb_ref[...], w_ref[...], preferred_element_type=jnp.float32)
        t_ref[...] = _pack_halves(r)

    return pl.pallas_call(
        body,
        grid=(NB // blk,),
        in_specs=[pl.BlockSpec((blk, DB), lambda i: (i, 0)),
                  pl.BlockSpec((DB, HP), lambda i: (0, 0))],
        out_specs=pl.BlockSpec((blk, HW), lambda i: (i, 0)),
        out_shape=jax.ShapeDtypeStruct((NB, HW), jnp.int32),
    )(fbonds_b, W_i_b)


def _tc_update(fbonds_b, W_i_b, nei_c, W_h_p, row0, nrows, acc):
    blk = 640
    off = row0 // blk

    def body(fb_ref, wi_ref, n_ref, w_ref, *rest):
        t_ref = rest[-1]
        nei_b = _unpack_halves(n_ref[...]).astype(jnp.bfloat16)
        r = (jnp.dot(fb_ref[...], wi_ref[...],
                     preferred_element_type=jnp.float32)
             + jnp.dot(nei_b, w_ref[...],
                       preferred_element_type=jnp.float32))
        t_ref[...] = _pack_halves(r)

    in_specs = [pl.BlockSpec((blk, DB), lambda i: (i + off, 0)),
                pl.BlockSpec((DB, HP), lambda i: (0, 0)),
                pl.BlockSpec((blk, HW), lambda i: (i, 0)),
                pl.BlockSpec((HP, HP), lambda i: (0, 0))]
    args = [fbonds_b, W_i_b, nei_c, W_h_p]
    aliases = {}
    if acc is not None:
        in_specs.append(pl.BlockSpec(memory_space=pl.ANY))
        args.append(acc)
        aliases = {4: 0}
    return pl.pallas_call(
        body,
        grid=(nrows // blk,),
        in_specs=in_specs,
        out_specs=pl.BlockSpec((blk, HW), lambda i: (i + off, 0)),
        out_shape=jax.ShapeDtypeStruct((NB, HW), jnp.int32),
        input_output_aliases=aliases,
    )(*args)


def _tc_readout(fatoms, nei_a, mol3d, Wo_a, Wo_n):
    blk = 1000

    def body(fat_ref, nei_ref, mol_ref, wa_ref, wn_ref, sum_ref, cnt_ref):
        i = pl.program_id(0)
        nei_b = _unpack_halves(nei_ref[...]).astype(jnp.bfloat16)
        ah = jnp.maximum(
            jnp.dot(fat_ref[...], wa_ref[...],
                    preferred_element_type=jnp.float32)
            + jnp.dot(nei_b, wn_ref[...],
                      preferred_element_type=jnp.float32), 0.0)
        mids = mol_ref[0, 0, :]
        mask = (mids[:, None]
                == lax.broadcasted_iota(jnp.int32, (blk, B), 1)
                ).astype(jnp.float32)
        psum = lax.dot_general(mask, ah, (((0,), (0,)), ((), ())),
                               preferred_element_type=jnp.float32)
        pcnt = jnp.sum(mask, axis=0)[:, None]

        @pl.when(i == 0)
        def _():
            sum_ref[...] = jnp.zeros_like(sum_ref)
            cnt_ref[...] = jnp.zeros_like(cnt_ref)

        sum_ref[...] += psum
        cnt_ref[...] += pcnt

    return pl.pallas_call(
        body,
        grid=(NA // blk,),
        in_specs=[pl.BlockSpec((blk, DA), lambda i: (i, 0)),
                  pl.BlockSpec((blk, HW), lambda i: (i, 0)),
                  pl.BlockSpec((1, 1, blk), lambda i: (i, 0, 0)),
                  pl.BlockSpec((DA, HP), lambda i: (0, 0)),
                  pl.BlockSpec((HP, HP), lambda i: (0, 0))],
        out_specs=[pl.BlockSpec((B, HP), lambda i: (0, 0)),
                   pl.BlockSpec((B, 1), lambda i: (0, 0))],
        out_shape=[jax.ShapeDtypeStruct((B, HP), jnp.float32),
                   jax.ShapeDtypeStruct((B, 1), jnp.float32)],
    )(fatoms, nei_a, mol3d, Wo_a, Wo_n)


def _tc_tail(sums, cnts, Gm_w_p, Gm_b_p, Gv_w_p, Gv_b_p,
             F1_w_p, F1_b_p, F2_w_p, F2_b_p, features, beta_arr):
    def body(s_ref, c_ref, gmw, gmb, gvw, gvb, f1w, f1b, f2w, f2b,
             feat, beta_ref, o_ref):
        cnt = jnp.maximum(c_ref[...], 1.0)
        mv = s_ref[...] / cnt
        zm = jnp.dot(mv, gmw[...], preferred_element_type=jnp.float32) + gmb[...]
        zlv = -jnp.abs(
            jnp.dot(mv, gvw[...], preferred_element_type=jnp.float32) + gvb[...])
        kl = -0.5 * jnp.sum(1.0 + zlv - zm * zm - jnp.exp(zlv)) / B
        h1 = jnp.maximum(
            jnp.dot(mv, f1w[...], preferred_element_type=jnp.float32)
            + f1b[...], 0.0)
        pred = jnp.dot(h1, f2w[...], preferred_element_type=jnp.float32) + f2b[...]
        dp = pred[:, 0:1] - feat[...]
        prop = jnp.mean(dp * dp)
        o_ref[...] = prop + beta_ref[...] * kl

    return pl.pallas_call(
        body,
        out_shape=jax.ShapeDtypeStruct((1, 1), jnp.float32),
    )(sums, cnts, Gm_w_p, Gm_b_p, Gv_w_p, Gv_b_p,
      F1_w_p, F1_b_p, F2_w_p, F2_b_p, features, beta_arr)


def kernel(fatoms, fbonds, agraph, bgraph, mol_ids, features, epsilon, beta,
           W_i, W_h, W_o, Gm_w, Gm_b, Gv_w, Gv_b, F1_w, F1_b, F2_w, F2_b):
    f32 = jnp.float32
    bf16 = jnp.bfloat16
    W_i_b = jnp.pad(W_i, ((0, 0), (0, HP - H))).astype(bf16)
    W_h_b = jnp.pad(W_h, ((0, HP - H), (0, HP - H))).astype(bf16)
    Wo_a = jnp.pad(W_o[:DA], ((0, 0), (0, HP - H))).astype(bf16)
    Wo_n = jnp.pad(W_o[DA:], ((0, HP - H), (0, HP - H))).astype(bf16)
    Gm_w_p = jnp.pad(Gm_w, ((0, HP - H), (0, LATP - LAT)))
    Gm_b_p = jnp.pad(Gm_b, ((0, LATP - LAT)))[None, :]
    Gv_w_p = jnp.pad(Gv_w, ((0, HP - H), (0, LATP - LAT)))
    Gv_b_p = jnp.pad(Gv_b, ((0, LATP - LAT)))[None, :]
    F1_w_p = jnp.pad(F1_w, ((0, HP - H), (0, HP - H)))
    F1_b_p = jnp.pad(F1_b, ((0, HP - H)))[None, :]
    F2_w_p = jnp.pad(F2_w, ((0, HP - H), (0, LATP - 1)))
    F2_b_p = jnp.pad(F2_b, ((0, LATP - 1)))[None, :]
    beta_arr = jnp.asarray(beta, f32).reshape(1, 1)

    bgraph_flat = bgraph.astype(jnp.int32).reshape(-1)
    pad_idx = (jnp.arange((NAP - NA) * MAXNB, dtype=jnp.int32)
               % NB).reshape(NAP - NA, MAXNB)
    agraph_flat = jnp.concatenate(
        [agraph.astype(jnp.int32), pad_idx], axis=0).reshape(-1)
    mol3d = mol_ids.astype(jnp.int32).reshape(NA // 1000, 1, 1000)

    fbonds_b = fbonds.astype(bf16)
    btable = _tc_binput(fbonds_b, W_i_b)
    sizes = (51200, 46080, 35840, 20480, 6400)
    offs = [sum(sizes[:c]) for c in range(len(sizes))]

    def mp_round(table):
        neis = [
            _sc_gather_relu_sum(table, bgraph_flat, offs[c], sizes[c],
                                sizes[c] // NW)
            for c in range(len(sizes))]
        acc = None
        for c in range(len(sizes)):
            acc = _tc_update(fbonds_b, W_i_b, neis[c], W_h_b,
                             offs[c], sizes[c], acc)
        return acc

    m1 = mp_round(btable)
    m2 = mp_round(m1)
    agraph_flat = agraph_flat + m1[0, 0] * 0
    nei_a = _sc_gather_relu_sum(m2, agraph_flat, 0, NAP, NAP // NW)
    sums, cnts = _tc_readout(fatoms.astype(bf16), nei_a, mol3d, Wo_a, Wo_n)
    out = _tc_tail(sums, cnts, Gm_w_p, Gm_b_p, Gv_w_p, Gv_b_p,
                   F1_w_p, F1_b_p, F2_w_p, F2_b_p, features, beta_arr)
    return out.reshape(())
